# baseline (speedup 1.0000x reference)
import functools

import jax
import jax.numpy as jnp
from jax import lax
from jax.experimental import pallas as pl
from jax.experimental.pallas import tpu as pltpu
from jax.experimental.pallas import tpu_sc as plsc

N = 10000
E = 320000
B = 128
F32 = jnp.float32

NC, NS = 2, 16
CH = 128
NP = 10240
SL = NP // NS
EP = 327680
EPH = EP // 2

_mesh = plsc.VectorSubcoreMesh(core_axis_name="c", subcore_axis_name="s")
_sc_params = pltpu.CompilerParams(use_tc_tiling_on_sc=False)

NB = 4
LAG = 2


def _pipelined_prop(tbl_hbm, acc, srcv_all, dstv_all, rows, gsems, ssems,
                    src_hbm, dst_hbm, base_row, nch):
    pltpu.sync_copy(src_hbm.at[pl.ds(base_row, nch)], srcv_all)
    pltpu.sync_copy(dst_hbm.at[pl.ds(base_row, nch)], dstv_all)

    @pl.loop(0, nch, step=NB)
    def _(i0):
        for b in range(NB):
            i = i0 + b

            @pl.when(i0 > 0)
            def _():
                pltpu.make_async_copy(
                    rows.at[b], acc.at[dstv_all.at[i - NB]], ssems[b]).wait()

            pltpu.async_copy(tbl_hbm.at[srcv_all.at[i]], rows.at[b], gsems[b])

            bs = (b - LAG) % NB
            j = i - LAG

            @pl.when(j >= 0)
            def _():
                pltpu.make_async_copy(
                    tbl_hbm.at[srcv_all.at[j]], rows.at[bs], gsems[bs]).wait()
                pltpu.async_copy(
                    rows.at[bs], acc.at[dstv_all.at[j]], ssems[bs], add=True)

    for k in range(LAG):
        j = nch - LAG + k
        bs = j % NB
        pltpu.make_async_copy(
            tbl_hbm.at[srcv_all.at[j]], rows.at[bs], gsems[bs]).wait()
        pltpu.async_copy(rows.at[bs], acc.at[dstv_all.at[j]], ssems[bs],
                         add=True)
    for b in range(NB):
        pltpu.make_async_copy(
            rows.at[b], acc.at[dstv_all.at[nch - NB + b]], ssems[b]).wait()


IB2 = 16


def _ring_block(tbl_hbm, acc, srcv, dstv, rows, gsems, ssems, ib):
    pltpu.async_copy(tbl_hbm.at[srcv.at[0]], rows.at[0], gsems[0])

    @pl.loop(0, ib, step=2)
    def _(c0):
        for b in range(2):
            c = c0 + b
            nxt = 1 - b
            if b == 0:
                @pl.when((c0 >= 1) & (c0 + 1 < ib))
                def _():
                    pltpu.make_async_copy(
                        rows.at[1], acc.at[dstv.at[c - 1]], ssems[1]).wait()
                    pltpu.async_copy(tbl_hbm.at[srcv.at[c + 1]], rows.at[1],
                                     gsems[1])

                @pl.when(c0 < 1)
                def _():
                    pltpu.async_copy(tbl_hbm.at[srcv.at[c + 1]], rows.at[1],
                                     gsems[1])
            else:
                @pl.when(c0 + 2 < ib)
                def _():
                    pltpu.make_async_copy(
                        rows.at[0], acc.at[dstv.at[c - 1]], ssems[0]).wait()
                    pltpu.async_copy(tbl_hbm.at[srcv.at[c + 1]], rows.at[0],
                                     gsems[0])

            pltpu.make_async_copy(
                tbl_hbm.at[srcv.at[c]], rows.at[b], gsems[b]).wait()
            pltpu.async_copy(rows.at[b], acc.at[dstv.at[c]], ssems[b],
                             add=True)

    pltpu.make_async_copy(rows.at[0], acc.at[dstv.at[ib - 2]],
                          ssems[0]).wait()
    pltpu.make_async_copy(rows.at[1], acc.at[dstv.at[ib - 1]],
                          ssems[1]).wait()


def _blocked_prop(tbl_hbm, acc, srcb, dstb, rows, gsems, ssems, isems,
                  src_hbm, dst_hbm, base_row, nblk, ib):
    def load_blk(k, slot):
        pltpu.async_copy(src_hbm.at[pl.ds(base_row + k * ib, ib)],
                         srcb.at[slot], isems[slot])
        pltpu.async_copy(dst_hbm.at[pl.ds(base_row + k * ib, ib)],
                         dstb.at[slot], isems[slot])

    def wait_blk(k, slot):
        pltpu.make_async_copy(src_hbm.at[pl.ds(base_row + k * ib, ib)],
                              srcb.at[slot], isems[slot]).wait()
        pltpu.make_async_copy(dst_hbm.at[pl.ds(base_row + k * ib, ib)],
                              dstb.at[slot], isems[slot]).wait()

    load_blk(0, 0)
    load_blk(1, 1)

    @pl.loop(0, nblk, step=2)
    def _(k0):
        for kb in range(2):
            k = k0 + kb
            wait_blk(k, kb)
            _ring_block(tbl_hbm, acc, srcb.at[kb], dstb.at[kb], rows,
                        gsems, ssems, ib)

            @pl.when(k + 2 < nblk)
            def _():
                load_blk(k + 2, kb)


def _zero_init(zeros_hbm, acc, s):
    pltpu.sync_copy(zeros_hbm, acc.at[pl.ds(s * SL, SL)])


def _writeback(acc, out_hbm, c, s):
    pltpu.sync_copy(acc.at[pl.ds(s * SL, SL)], out_hbm.at[c, pl.ds(s * SL, SL)])


NCH1 = EPH // NS // CH
NCH2 = EP // NS // CH
_SEMS4 = [pltpu.SemaphoreType.DMA] * NB


@functools.partial(
    pl.kernel,
    out_type=jax.ShapeDtypeStruct((NC, NP, 16), F32),
    mesh=_mesh,
    scratch_types=[
        pltpu.VMEM_SHARED((NP, 16), F32),
        pltpu.VMEM((CH, 16), F32),
        pltpu.VMEM((NCH1, CH), jnp.int32),
    ] + _SEMS4,
    compiler_params=_sc_params,
)
def _sc_deg(dst_hbm, ones_hbm, zeros_hbm, out_hbm, acc, ones_v, dstv_all,
            *ssems):
    c = lax.axis_index("c")
    s = lax.axis_index("s")
    pltpu.sync_copy(ones_hbm, ones_v)
    _zero_init(zeros_hbm, acc, s)
    plsc.subcore_barrier()
    base_row = c * (EPH // CH) + s * NCH1
    pltpu.sync_copy(dst_hbm.at[pl.ds(base_row, NCH1)], dstv_all)

    @pl.loop(0, NCH1, step=NB)
    def _(i0):
        for b in range(NB):
            i = i0 + b

            @pl.when(i0 > 0)
            def _():
                pltpu.make_async_copy(
                    ones_v, acc.at[dstv_all.at[i - NB]], ssems[b]).wait()

            pltpu.async_copy(ones_v, acc.at[dstv_all.at[i]], ssems[b],
                             add=True)

    for b in range(NB):
        pltpu.make_async_copy(
            ones_v, acc.at[dstv_all.at[NCH1 - NB + b]], ssems[b]).wait()

    plsc.subcore_barrier()
    _writeback(acc, out_hbm, c, s)


@functools.partial(
    pl.kernel,
    out_type=jax.ShapeDtypeStruct((NC, NP, 16), F32),
    mesh=_mesh,
    scratch_types=[
        pltpu.VMEM_SHARED((NP, 16), F32),
        pltpu.VMEM((NB, CH, 16), F32),
        pltpu.VMEM((NCH1, CH), jnp.int32),
        pltpu.VMEM((NCH1, CH), jnp.int32),
    ] + _SEMS4 + _SEMS4,
    compiler_params=_sc_params,
)
def _sc_prop1(src_hbm, dst_hbm, tbl_hbm, zeros_hbm, out_hbm, acc,
              rows, srcv_all, dstv_all, *sems):
    c = lax.axis_index("c")
    s = lax.axis_index("s")
    _zero_init(zeros_hbm, acc, s)
    plsc.subcore_barrier()
    base_row = c * (EPH // CH) + s * NCH1
    _pipelined_prop(tbl_hbm, acc, srcv_all, dstv_all, rows,
                    sems[:NB], sems[NB:], src_hbm, dst_hbm, base_row, NCH1)
    plsc.subcore_barrier()
    _writeback(acc, out_hbm, c, s)


@functools.partial(
    pl.kernel,
    out_type=jax.ShapeDtypeStruct((NC, NP, 128), F32),
    mesh=_mesh,
    scratch_types=[
        pltpu.VMEM_SHARED((NP, 128), F32),
        pltpu.VMEM((2, CH, 128), F32),
        pltpu.VMEM((2, IB2, CH), jnp.int32),
        pltpu.VMEM((2, IB2, CH), jnp.int32),
        pltpu.SemaphoreType.DMA,
        pltpu.SemaphoreType.DMA,
        pltpu.SemaphoreType.DMA,
        pltpu.SemaphoreType.DMA,
        pltpu.SemaphoreType.DMA,
        pltpu.SemaphoreType.DMA,
    ],
)
def _sc_prop2(src_hbm, dst_hbm, tblA_hbm, tblB_hbm, zeros_hbm, out_hbm,
              acc, rows, srcb, dstb, g0, g1, s0, s1, i0, i1):
    c = lax.axis_index("c")
    s = lax.axis_index("s")
    _zero_init(zeros_hbm, acc, s)
    plsc.subcore_barrier()
    base_row = s * NCH2
    nblk = NCH2 // IB2

    @pl.when(c == 0)
    def _():
        _blocked_prop(tblA_hbm, acc, srcb, dstb, rows, [g0, g1], [s0, s1],
                      [i0, i1], src_hbm, dst_hbm, base_row, nblk, IB2)

    @pl.when(c == 1)
    def _():
        _blocked_prop(tblB_hbm, acc, srcb, dstb, rows, [g0, g1], [s0, s1],
                      [i0, i1], src_hbm, dst_hbm, base_row, nblk, IB2)

    plsc.subcore_barrier()
    _writeback(acc, out_hbm, c, s)


def _tc_prescale_body(degp_ref, x_ref, dinv_ref, xs_ref):
    deg = degp_ref[0, :N, 0] + degp_ref[1, :N, 0] + 1.0
    dinv = lax.rsqrt(deg)[:, None]
    dinv_ref[...] = dinv
    xs = x_ref[...] * dinv
    xs_ref[...] = jnp.pad(xs, ((0, NP - N), (0, 12)))


def _tc_prescale(degp, x):
    return pl.pallas_call(
        _tc_prescale_body,
        out_shape=(jax.ShapeDtypeStruct((N, 1), F32),
                   jax.ShapeDtypeStruct((NP, 16), F32)),
    )(degp, x)


def _tc_hidden_body(p1_ref, xs_ref, dinv_ref, W1_ref, b1_ref, W2_ref,
                    tblA_ref, tblB_ref):
    dinv = dinv_ref[...]
    px = dinv * (p1_ref[0, :N, 0:4] + p1_ref[1, :N, 0:4] + xs_ref[:N, 0:4])
    h1 = jax.nn.relu(
        jnp.dot(px, W1_ref[...], preferred_element_type=F32) + b1_ref[...])
    g = jnp.dot(h1, W2_ref[...], preferred_element_type=F32)
    gs = g * dinv
    pad = ((0, NP - N), (0, 0))
    tblA_ref[...] = jnp.pad(gs[:, :128], pad)
    tblB_ref[...] = jnp.pad(gs[:, 128:], pad)


def _tc_hidden(p1, xs, dinv, W1, b1, W2):
    return pl.pallas_call(
        _tc_hidden_body,
        out_shape=(jax.ShapeDtypeStruct((NP, 128), F32),
                   jax.ShapeDtypeStruct((NP, 128), F32)),
    )(p1, xs, dinv, W1, b1, W2)


def _bn(h, g, b, eps=1e-5):
    m = jnp.mean(h, axis=0)
    v = jnp.var(h, axis=0)
    return (h - m) * lax.rsqrt(v + eps) * g + b


def _elu(h):
    return jnp.where(h > 0, h, jnp.exp(h) - 1.0)


def _tc_head_body(p2_ref, tblA_ref, tblB_ref, dinv_ref, b2_ref, batchT_ref,
                  drug_ref,
                  cW1_ref, cb1_ref, cg1_ref, cbe1_ref, cW2_ref, cb2_ref,
                  dW1_ref, db1_ref, dg1_ref, dbe1_ref, dW2_ref, db2_ref,
                  dg2_ref, dbe2_ref, fW1_ref, fb1_ref, fg1_ref, fbe1_ref,
                  fW2_ref, fb2_ref, fg2_ref, fbe2_ref, fW3_ref, fb3_ref,
                  y_ref):
    dinv = dinv_ref[...]
    agg = jnp.concatenate(
        [p2_ref[0, :N, :] + tblA_ref[:N, :],
         p2_ref[1, :N, :] + tblB_ref[:N, :]], axis=1)
    h2 = jax.nn.relu(dinv * agg + b2_ref[...])

    cols = batchT_ref[...]
    rows = lax.broadcasted_iota(jnp.int32, (B, N), 0)
    maskT = (cols == rows).astype(F32)
    cnt = jnp.sum(maskT, axis=1)
    ssum = jnp.dot(maskT, h2, preferred_element_type=F32)
    pooled = ssum / jnp.maximum(cnt, 1.0)[:, None]

    dh = jax.nn.relu(_bn(
        jnp.dot(drug_ref[...], dW1_ref[...], preferred_element_type=F32)
        + db1_ref[...], dg1_ref[...], dbe1_ref[...]))
    dh = jax.nn.relu(_bn(
        jnp.dot(dh, dW2_ref[...], preferred_element_type=F32)
        + db2_ref[...], dg2_ref[...], dbe2_ref[...]))

    ch = jax.nn.relu(_bn(
        jnp.dot(pooled, cW1_ref[...], preferred_element_type=F32)
        + cb1_ref[...], cg1_ref[...], cbe1_ref[...]))
    ch = jax.nn.relu(
        jnp.dot(ch, cW2_ref[...], preferred_element_type=F32) + cb2_ref[...])

    cat = jnp.concatenate([ch, dh], axis=-1)
    f = _elu(_bn(
        jnp.dot(cat, fW1_ref[...], preferred_element_type=F32)
        + fb1_ref[...], fg1_ref[...], fbe1_ref[...]))
    f = _elu(_bn(
        jnp.dot(f, fW2_ref[...], preferred_element_type=F32)
        + fb2_ref[...], fg2_ref[...], fbe2_ref[...]))
    y_ref[...] = jnp.dot(f, fW3_ref[...], preferred_element_type=F32) \
        + fb3_ref[...]


def _tc_head(p2, tblA, tblB, dinv, b2, batchT, drug, args):
    return pl.pallas_call(
        _tc_head_body,
        out_shape=jax.ShapeDtypeStruct((B, 1), F32),
    )(p2, tblA, tblB, dinv, b2, batchT, drug, *args)


def kernel(x, edge_index, batch, drug, W1, b1, W2, b2, cW1, cb1, cg1, cbe1,
           cW2, cb2, dW1, db1, dg1, dbe1, dW2, db2, dg2, dbe2, fW1, fb1,
           fg1, fbe1, fW2, fb2, fg2, fbe2, fW3, fb3):
    padi = (N + (jnp.arange(EP - E, dtype=jnp.int32) % 128)).astype(jnp.int32)
    src = jnp.concatenate(
        [edge_index[0].astype(jnp.int32), padi]).reshape(EP // CH, CH)
    dst = jnp.concatenate(
        [edge_index[1].astype(jnp.int32), padi]).reshape(EP // CH, CH)

    ones16 = jnp.ones((CH, 16), F32)
    zeros16 = jnp.zeros((SL, 16), F32)
    zeros128 = jnp.zeros((SL, 128), F32)

    degp = _sc_deg(dst, ones16, zeros16)
    dinv, xs = _tc_prescale(degp, x)
    p1 = _sc_prop1(src, dst, xs, zeros16)
    tblA, tblB = _tc_hidden(p1, xs, dinv, W1, b1, W2)
    p2 = _sc_prop2(src, dst, tblA, tblB, zeros128)

    batchT = batch.reshape(1, N).astype(jnp.int32)
    args = (cW1, cb1, cg1, cbe1, cW2, cb2, dW1, db1, dg1, dbe1, dW2, db2,
            dg2, dbe2, fW1, fb1, fg1, fbe1, fW2, fb2, fg2, fbe2, fW3, fb3)
    y = _tc_head(p2, tblA, tblB, dinv, b2, batchT, drug, args)
    return y.reshape(B)


# speedup vs baseline: 1.0143x; 1.0143x over previous
import functools

import jax
import jax.numpy as jnp
from jax import lax
from jax.experimental import pallas as pl
from jax.experimental.pallas import tpu as pltpu
from jax.experimental.pallas import tpu_sc as plsc

N = 10000
E = 320000
B = 128
F32 = jnp.float32

NC, NS = 2, 16
CH = 128
NP = 10240
SL = NP // NS
EP = 327680
EPH = EP // 2

_mesh = plsc.VectorSubcoreMesh(core_axis_name="c", subcore_axis_name="s")
_sc_params = pltpu.CompilerParams(use_tc_tiling_on_sc=False)

NB = 4
LAG = 2


def _pipelined_prop(tbl_hbm, acc, srcv_all, dstv_all, rows, gsems, ssems,
                    src_hbm, dst_hbm, base_row, nch):
    pltpu.sync_copy(src_hbm.at[pl.ds(base_row, nch)], srcv_all)
    pltpu.sync_copy(dst_hbm.at[pl.ds(base_row, nch)], dstv_all)

    @pl.loop(0, nch, step=NB)
    def _(i0):
        for b in range(NB):
            i = i0 + b

            @pl.when(i0 > 0)
            def _():
                pltpu.make_async_copy(
                    rows.at[b], acc.at[dstv_all.at[i - NB]], ssems[b]).wait()

            pltpu.async_copy(tbl_hbm.at[srcv_all.at[i]], rows.at[b], gsems[b])

            bs = (b - LAG) % NB
            j = i - LAG

            @pl.when(j >= 0)
            def _():
                pltpu.make_async_copy(
                    tbl_hbm.at[srcv_all.at[j]], rows.at[bs], gsems[bs]).wait()
                pltpu.async_copy(
                    rows.at[bs], acc.at[dstv_all.at[j]], ssems[bs], add=True)

    for k in range(LAG):
        j = nch - LAG + k
        bs = j % NB
        pltpu.make_async_copy(
            tbl_hbm.at[srcv_all.at[j]], rows.at[bs], gsems[bs]).wait()
        pltpu.async_copy(rows.at[bs], acc.at[dstv_all.at[j]], ssems[bs],
                         add=True)
    for b in range(NB):
        pltpu.make_async_copy(
            rows.at[b], acc.at[dstv_all.at[nch - NB + b]], ssems[b]).wait()


IB2 = 16


def _ring_block(tbl_hbm, acc, srcv, dstv, rows, gsems, ssems, ib):
    pltpu.async_copy(tbl_hbm.at[srcv.at[0]], rows.at[0], gsems[0])

    @pl.loop(0, ib, step=2)
    def _(c0):
        for b in range(2):
            c = c0 + b
            nxt = 1 - b
            if b == 0:
                @pl.when((c0 >= 1) & (c0 + 1 < ib))
                def _():
                    pltpu.make_async_copy(
                        rows.at[1], acc.at[dstv.at[c - 1]], ssems[1]).wait()
                    pltpu.async_copy(tbl_hbm.at[srcv.at[c + 1]], rows.at[1],
                                     gsems[1])

                @pl.when(c0 < 1)
                def _():
                    pltpu.async_copy(tbl_hbm.at[srcv.at[c + 1]], rows.at[1],
                                     gsems[1])
            else:
                @pl.when(c0 + 2 < ib)
                def _():
                    pltpu.make_async_copy(
                        rows.at[0], acc.at[dstv.at[c - 1]], ssems[0]).wait()
                    pltpu.async_copy(tbl_hbm.at[srcv.at[c + 1]], rows.at[0],
                                     gsems[0])

            pltpu.make_async_copy(
                tbl_hbm.at[srcv.at[c]], rows.at[b], gsems[b]).wait()
            pltpu.async_copy(rows.at[b], acc.at[dstv.at[c]], ssems[b],
                             add=True)

    pltpu.make_async_copy(rows.at[0], acc.at[dstv.at[ib - 2]],
                          ssems[0]).wait()
    pltpu.make_async_copy(rows.at[1], acc.at[dstv.at[ib - 1]],
                          ssems[1]).wait()


def _blocked_prop(tbl_hbm, acc, srcb, dstb, rows, gsems, ssems, isems,
                  src_hbm, dst_hbm, base_row, nblk, ib):
    def load_blk(k, slot):
        pltpu.async_copy(src_hbm.at[pl.ds(base_row + k * ib, ib)],
                         srcb.at[slot], isems[slot])
        pltpu.async_copy(dst_hbm.at[pl.ds(base_row + k * ib, ib)],
                         dstb.at[slot], isems[slot])

    def wait_blk(k, slot):
        pltpu.make_async_copy(src_hbm.at[pl.ds(base_row + k * ib, ib)],
                              srcb.at[slot], isems[slot]).wait()
        pltpu.make_async_copy(dst_hbm.at[pl.ds(base_row + k * ib, ib)],
                              dstb.at[slot], isems[slot]).wait()

    load_blk(0, 0)
    load_blk(1, 1)

    @pl.loop(0, nblk, step=2)
    def _(k0):
        for kb in range(2):
            k = k0 + kb
            wait_blk(k, kb)
            _ring_block(tbl_hbm, acc, srcb.at[kb], dstb.at[kb], rows,
                        gsems, ssems, ib)

            @pl.when(k + 2 < nblk)
            def _():
                load_blk(k + 2, kb)


def _zero_init(zeros_hbm, acc, s):
    pltpu.sync_copy(zeros_hbm, acc.at[pl.ds(s * SL, SL)])


def _writeback(acc, out_hbm, c, s):
    pltpu.sync_copy(acc.at[pl.ds(s * SL, SL)], out_hbm.at[c, pl.ds(s * SL, SL)])


NCH1 = EPH // NS // CH
NCH2 = EP // NS // CH
_SEMS4 = [pltpu.SemaphoreType.DMA] * NB


@functools.partial(
    pl.kernel,
    out_type=jax.ShapeDtypeStruct((NC, NP, 16), F32),
    mesh=_mesh,
    scratch_types=[
        pltpu.VMEM_SHARED((NP, 16), F32),
        pltpu.VMEM((CH, 16), F32),
        pltpu.VMEM((NCH1, CH), jnp.int32),
    ] + _SEMS4,
    compiler_params=_sc_params,
)
def _sc_deg(edges_hbm, ones_hbm, zeros_hbm, out_hbm, acc, ones_v, dstv_all,
            *ssems):
    c = lax.axis_index("c")
    s = lax.axis_index("s")
    pltpu.sync_copy(ones_hbm, ones_v)
    _zero_init(zeros_hbm, acc, s)
    plsc.subcore_barrier()
    base_row = c * (EPH // CH) + s * NCH1
    pltpu.sync_copy(edges_hbm.at[1, pl.ds(base_row, NCH1)], dstv_all)

    @pl.loop(0, NCH1, step=NB)
    def _(i0):
        for b in range(NB):
            i = i0 + b

            @pl.when(i0 > 0)
            def _():
                pltpu.make_async_copy(
                    ones_v, acc.at[dstv_all.at[i - NB]], ssems[b]).wait()

            pltpu.async_copy(ones_v, acc.at[dstv_all.at[i]], ssems[b],
                             add=True)

    for b in range(NB):
        pltpu.make_async_copy(
            ones_v, acc.at[dstv_all.at[NCH1 - NB + b]], ssems[b]).wait()

    plsc.subcore_barrier()
    _writeback(acc, out_hbm, c, s)


@functools.partial(
    pl.kernel,
    out_type=jax.ShapeDtypeStruct((NC, NP, 16), F32),
    mesh=_mesh,
    scratch_types=[
        pltpu.VMEM_SHARED((NP, 16), F32),
        pltpu.VMEM((NB, CH, 16), F32),
        pltpu.VMEM((NCH1, CH), jnp.int32),
        pltpu.VMEM((NCH1, CH), jnp.int32),
    ] + _SEMS4 + _SEMS4,
    compiler_params=_sc_params,
)
def _sc_prop1(edges_hbm, tbl_hbm, zeros_hbm, out_hbm, acc,
              rows, srcv_all, dstv_all, *sems):
    c = lax.axis_index("c")
    s = lax.axis_index("s")
    _zero_init(zeros_hbm, acc, s)
    plsc.subcore_barrier()
    base_row = c * (EPH // CH) + s * NCH1
    _pipelined_prop(tbl_hbm, acc, srcv_all, dstv_all, rows,
                    sems[:NB], sems[NB:], edges_hbm.at[0], edges_hbm.at[1],
                    base_row, NCH1)
    plsc.subcore_barrier()
    _writeback(acc, out_hbm, c, s)


@functools.partial(
    pl.kernel,
    out_type=jax.ShapeDtypeStruct((NC, NP, 128), F32),
    mesh=_mesh,
    scratch_types=[
        pltpu.VMEM_SHARED((NP, 128), F32),
        pltpu.VMEM((2, CH, 128), F32),
        pltpu.VMEM((2, IB2, CH), jnp.int32),
        pltpu.VMEM((2, IB2, CH), jnp.int32),
        pltpu.SemaphoreType.DMA,
        pltpu.SemaphoreType.DMA,
        pltpu.SemaphoreType.DMA,
        pltpu.SemaphoreType.DMA,
        pltpu.SemaphoreType.DMA,
        pltpu.SemaphoreType.DMA,
    ],
)
def _sc_prop2(edges_hbm, tblA_hbm, tblB_hbm, zeros_hbm, out_hbm,
              acc, rows, srcb, dstb, g0, g1, s0, s1, i0, i1):
    c = lax.axis_index("c")
    s = lax.axis_index("s")
    _zero_init(zeros_hbm, acc, s)
    plsc.subcore_barrier()
    base_row = s * NCH2
    nblk = NCH2 // IB2

    @pl.when(c == 0)
    def _():
        _blocked_prop(tblA_hbm, acc, srcb, dstb, rows, [g0, g1], [s0, s1],
                      [i0, i1], edges_hbm.at[0], edges_hbm.at[1], base_row,
                      nblk, IB2)

    @pl.when(c == 1)
    def _():
        _blocked_prop(tblB_hbm, acc, srcb, dstb, rows, [g0, g1], [s0, s1],
                      [i0, i1], edges_hbm.at[0], edges_hbm.at[1], base_row,
                      nblk, IB2)

    plsc.subcore_barrier()
    _writeback(acc, out_hbm, c, s)


def _tc_hidden_body(p1_ref, xs_ref, dinv_ref, W1_ref, b1_ref, W2_ref,
                    tblA_ref, tblB_ref):
    dinv = dinv_ref[...]
    px = dinv * (p1_ref[0, :N, 0:4] + p1_ref[1, :N, 0:4] + xs_ref[:N, 0:4])
    h1 = jax.nn.relu(
        jnp.dot(px, W1_ref[...], preferred_element_type=F32) + b1_ref[...])
    g = jnp.dot(h1, W2_ref[...], preferred_element_type=F32)
    gs = g * dinv
    pad = ((0, NP - N), (0, 0))
    tblA_ref[...] = jnp.pad(gs[:, :128], pad)
    tblB_ref[...] = jnp.pad(gs[:, 128:], pad)


def _tc_hidden(p1, xs, dinv, W1, b1, W2):
    return pl.pallas_call(
        _tc_hidden_body,
        out_shape=(jax.ShapeDtypeStruct((NP, 128), F32),
                   jax.ShapeDtypeStruct((NP, 128), F32)),
    )(p1, xs, dinv, W1, b1, W2)


def _bn(h, g, b, eps=1e-5):
    m = jnp.mean(h, axis=0)
    v = jnp.var(h, axis=0)
    return (h - m) * lax.rsqrt(v + eps) * g + b


def _elu(h):
    return jnp.where(h > 0, h, jnp.exp(h) - 1.0)


def _tc_head_body(p2_ref, tblA_ref, tblB_ref, dinv_ref, b2_ref, batchT_ref,
                  drug_ref,
                  cW1_ref, cb1_ref, cg1_ref, cbe1_ref, cW2_ref, cb2_ref,
                  dW1_ref, db1_ref, dg1_ref, dbe1_ref, dW2_ref, db2_ref,
                  dg2_ref, dbe2_ref, fW1_ref, fb1_ref, fg1_ref, fbe1_ref,
                  fW2_ref, fb2_ref, fg2_ref, fbe2_ref, fW3_ref, fb3_ref,
                  y_ref):
    dinv = dinv_ref[...]
    agg = jnp.concatenate(
        [p2_ref[0, :N, :] + tblA_ref[:N, :],
         p2_ref[1, :N, :] + tblB_ref[:N, :]], axis=1)
    h2 = jax.nn.relu(dinv * agg + b2_ref[...])

    cols = batchT_ref[...]
    rows = lax.broadcasted_iota(jnp.int32, (B, N), 0)
    maskT = (cols == rows).astype(F32)
    cnt = jnp.sum(maskT, axis=1)
    ssum = jnp.dot(maskT, h2, preferred_element_type=F32)
    pooled = ssum / jnp.maximum(cnt, 1.0)[:, None]

    dh = jax.nn.relu(_bn(
        jnp.dot(drug_ref[...], dW1_ref[...], preferred_element_type=F32)
        + db1_ref[...], dg1_ref[...], dbe1_ref[...]))
    dh = jax.nn.relu(_bn(
        jnp.dot(dh, dW2_ref[...], preferred_element_type=F32)
        + db2_ref[...], dg2_ref[...], dbe2_ref[...]))

    ch = jax.nn.relu(_bn(
        jnp.dot(pooled, cW1_ref[...], preferred_element_type=F32)
        + cb1_ref[...], cg1_ref[...], cbe1_ref[...]))
    ch = jax.nn.relu(
        jnp.dot(ch, cW2_ref[...], preferred_element_type=F32) + cb2_ref[...])

    cat = jnp.concatenate([ch, dh], axis=-1)
    f = _elu(_bn(
        jnp.dot(cat, fW1_ref[...], preferred_element_type=F32)
        + fb1_ref[...], fg1_ref[...], fbe1_ref[...]))
    f = _elu(_bn(
        jnp.dot(f, fW2_ref[...], preferred_element_type=F32)
        + fb2_ref[...], fg2_ref[...], fbe2_ref[...]))
    y_ref[...] = jnp.dot(f, fW3_ref[...], preferred_element_type=F32) \
        + fb3_ref[...]


def _tc_head(p2, tblA, tblB, dinv, b2, batchT, drug, args):
    return pl.pallas_call(
        _tc_head_body,
        out_shape=jax.ShapeDtypeStruct((B, 1), F32),
    )(p2, tblA, tblB, dinv, b2, batchT, drug, *args)


def kernel(x, edge_index, batch, drug, W1, b1, W2, b2, cW1, cb1, cg1, cbe1,
           cW2, cb2, dW1, db1, dg1, dbe1, dW2, db2, dg2, dbe2, fW1, fb1,
           fg1, fbe1, fW2, fb2, fg2, fbe2, fW3, fb3):
    padi = (N + (jnp.arange(EP - E, dtype=jnp.int32) % 128)).astype(jnp.int32)
    padi2 = jnp.broadcast_to(padi, (2, EP - E))
    edges = jnp.concatenate(
        [edge_index.astype(jnp.int32), padi2], axis=1).reshape(2, EP // CH, CH)

    ones16 = jnp.ones((CH, 16), F32)
    zeros16 = jnp.zeros((SL, 16), F32)
    zeros128 = jnp.zeros((SL, 128), F32)

    degp = _sc_deg(edges, ones16, zeros16)
    deg = degp[0, :N, 0] + degp[1, :N, 0] + 1.0
    dinv = lax.rsqrt(deg)[:, None]
    xs = jnp.pad(x * dinv, ((0, NP - N), (0, 12)))
    p1 = _sc_prop1(edges, xs, zeros16)
    tblA, tblB = _tc_hidden(p1, xs, dinv, W1, b1, W2)
    p2 = _sc_prop2(edges, tblA, tblB, zeros128)

    batchT = batch.reshape(1, N).astype(jnp.int32)
    args = (cW1, cb1, cg1, cbe1, cW2, cb2, dW1, db1, dg1, dbe1, dW2, db2,
            dg2, dbe2, fW1, fb1, fg1, fbe1, fW2, fb2, fg2, fbe2, fW3, fb3)
    y = _tc_head(p2, tblA, tblB, dinv, b2, batchT, drug, args)
    return y.reshape(B)


# speedup vs baseline: 1.0543x; 1.0395x over previous
import functools

import jax
import jax.numpy as jnp
from jax import lax
from jax.experimental import pallas as pl
from jax.experimental.pallas import tpu as pltpu
from jax.experimental.pallas import tpu_sc as plsc

N = 10000
E = 320000
B = 128
F32 = jnp.float32

NC, NS = 2, 16
CH = 128
NP = 10240
SL = NP // NS
EP = 327680
EPH = EP // 2

_mesh = plsc.VectorSubcoreMesh(core_axis_name="c", subcore_axis_name="s")
_sc_params = pltpu.CompilerParams(use_tc_tiling_on_sc=False)
_sc_params_nl = pltpu.CompilerParams(use_tc_tiling_on_sc=False,
                                     needs_layout_passes=False)

NB = 4
LAG = 2


def _pipelined_prop(tbl_hbm, acc, srcv_all, dstv_all, rows, gsems, ssems,
                    src_hbm, dst_hbm, base_row, nch):
    pltpu.sync_copy(src_hbm.at[pl.ds(base_row, nch)], srcv_all)
    pltpu.sync_copy(dst_hbm.at[pl.ds(base_row, nch)], dstv_all)

    @pl.loop(0, nch, step=NB)
    def _(i0):
        for b in range(NB):
            i = i0 + b

            @pl.when(i0 > 0)
            def _():
                pltpu.make_async_copy(
                    rows.at[b], acc.at[dstv_all.at[i - NB]], ssems[b]).wait()

            pltpu.async_copy(tbl_hbm.at[srcv_all.at[i]], rows.at[b], gsems[b])

            bs = (b - LAG) % NB
            j = i - LAG

            @pl.when(j >= 0)
            def _():
                pltpu.make_async_copy(
                    tbl_hbm.at[srcv_all.at[j]], rows.at[bs], gsems[bs]).wait()
                pltpu.async_copy(
                    rows.at[bs], acc.at[dstv_all.at[j]], ssems[bs], add=True)

    for k in range(LAG):
        j = nch - LAG + k
        bs = j % NB
        pltpu.make_async_copy(
            tbl_hbm.at[srcv_all.at[j]], rows.at[bs], gsems[bs]).wait()
        pltpu.async_copy(rows.at[bs], acc.at[dstv_all.at[j]], ssems[bs],
                         add=True)
    for b in range(NB):
        pltpu.make_async_copy(
            rows.at[b], acc.at[dstv_all.at[nch - NB + b]], ssems[b]).wait()


IB2 = 16


def _ring_block(tbl_hbm, acc, srcv, dstv, rows, gsems, ssems, ib):
    pltpu.async_copy(tbl_hbm.at[srcv.at[0]], rows.at[0], gsems[0])

    @pl.loop(0, ib, step=2)
    def _(c0):
        for b in range(2):
            c = c0 + b
            nxt = 1 - b
            if b == 0:
                @pl.when((c0 >= 1) & (c0 + 1 < ib))
                def _():
                    pltpu.make_async_copy(
                        rows.at[1], acc.at[dstv.at[c - 1]], ssems[1]).wait()
                    pltpu.async_copy(tbl_hbm.at[srcv.at[c + 1]], rows.at[1],
                                     gsems[1])

                @pl.when(c0 < 1)
                def _():
                    pltpu.async_copy(tbl_hbm.at[srcv.at[c + 1]], rows.at[1],
                                     gsems[1])
            else:
                @pl.when(c0 + 2 < ib)
                def _():
                    pltpu.make_async_copy(
                        rows.at[0], acc.at[dstv.at[c - 1]], ssems[0]).wait()
                    pltpu.async_copy(tbl_hbm.at[srcv.at[c + 1]], rows.at[0],
                                     gsems[0])

            pltpu.make_async_copy(
                tbl_hbm.at[srcv.at[c]], rows.at[b], gsems[b]).wait()
            pltpu.async_copy(rows.at[b], acc.at[dstv.at[c]], ssems[b],
                             add=True)

    pltpu.make_async_copy(rows.at[0], acc.at[dstv.at[ib - 2]],
                          ssems[0]).wait()
    pltpu.make_async_copy(rows.at[1], acc.at[dstv.at[ib - 1]],
                          ssems[1]).wait()


def _blocked_prop(tbl_hbm, acc, srcb, dstb, rows, gsems, ssems, isems,
                  src_hbm, dst_hbm, base_row, nblk, ib):
    def load_blk(k, slot):
        pltpu.async_copy(src_hbm.at[pl.ds(base_row + k * ib, ib)],
                         srcb.at[slot], isems[slot])
        pltpu.async_copy(dst_hbm.at[pl.ds(base_row + k * ib, ib)],
                         dstb.at[slot], isems[slot])

    def wait_blk(k, slot):
        pltpu.make_async_copy(src_hbm.at[pl.ds(base_row + k * ib, ib)],
                              srcb.at[slot], isems[slot]).wait()
        pltpu.make_async_copy(dst_hbm.at[pl.ds(base_row + k * ib, ib)],
                              dstb.at[slot], isems[slot]).wait()

    load_blk(0, 0)
    load_blk(1, 1)

    @pl.loop(0, nblk, step=2)
    def _(k0):
        for kb in range(2):
            k = k0 + kb
            wait_blk(k, kb)
            _ring_block(tbl_hbm, acc, srcb.at[kb], dstb.at[kb], rows,
                        gsems, ssems, ib)

            @pl.when(k + 2 < nblk)
            def _():
                load_blk(k + 2, kb)


def _zero_init(zeros_hbm, acc, s):
    pltpu.sync_copy(zeros_hbm, acc.at[pl.ds(s * SL, SL)])


def _writeback(acc, out_hbm, c, s):
    pltpu.sync_copy(acc.at[pl.ds(s * SL, SL)], out_hbm.at[c, pl.ds(s * SL, SL)])


NCH1 = EPH // NS // CH
NCH2 = EP // NS // CH
_SEMS4 = [pltpu.SemaphoreType.DMA] * NB


@functools.partial(
    pl.kernel,
    out_type=jax.ShapeDtypeStruct((NC, NP), F32),
    mesh=_mesh,
    scratch_types=[
        pltpu.VMEM_SHARED((NP, 16), F32),
        pltpu.VMEM((CH, 16), F32),
        pltpu.VMEM((NCH1, CH), jnp.int32),
        pltpu.VMEM((SL, 16), F32),
        pltpu.VMEM((SL,), F32),
    ] + _SEMS4,
    compiler_params=_sc_params_nl,
)
def _sc_deg(edges_hbm, ones_hbm, zeros_hbm, out_hbm, acc, ones_v, dstv_all,
            degbuf, packed, *ssems):
    c = lax.axis_index("c")
    s = lax.axis_index("s")
    pltpu.sync_copy(ones_hbm, ones_v)
    _zero_init(zeros_hbm, acc, s)
    plsc.subcore_barrier()
    base_row = c * (EPH // CH) + s * NCH1
    pltpu.sync_copy(edges_hbm.at[1, pl.ds(base_row, NCH1)], dstv_all)

    @pl.loop(0, NCH1, step=NB)
    def _(i0):
        for b in range(NB):
            i = i0 + b

            @pl.when(i0 > 0)
            def _():
                pltpu.make_async_copy(
                    ones_v, acc.at[dstv_all.at[i - NB]], ssems[b]).wait()

            pltpu.async_copy(ones_v, acc.at[dstv_all.at[i]], ssems[b],
                             add=True)

    for b in range(NB):
        pltpu.make_async_copy(
            ones_v, acc.at[dstv_all.at[NCH1 - NB + b]], ssems[b]).wait()

    plsc.subcore_barrier()
    pltpu.sync_copy(acc.at[pl.ds(s * SL, SL)], degbuf)
    lane = lax.iota(jnp.int32, 16)
    zero16 = lane * 0

    @pl.loop(0, SL // 16)
    def _(r0):
        vals = plsc.load_gather(degbuf, [r0 * 16 + lane, zero16])
        packed[pl.ds(r0 * 16, 16)] = vals

    pltpu.sync_copy(packed, out_hbm.at[c, pl.ds(s * SL, SL)])


@functools.partial(
    pl.kernel,
    out_type=jax.ShapeDtypeStruct((NC, NP, 16), F32),
    mesh=_mesh,
    scratch_types=[
        pltpu.VMEM_SHARED((NP, 16), F32),
        pltpu.VMEM((NB, CH, 16), F32),
        pltpu.VMEM((NCH1, CH), jnp.int32),
        pltpu.VMEM((NCH1, CH), jnp.int32),
    ] + _SEMS4 + _SEMS4,
    compiler_params=_sc_params,
)
def _sc_prop1(edges_hbm, tbl_hbm, zeros_hbm, out_hbm, acc,
              rows, srcv_all, dstv_all, *sems):
    c = lax.axis_index("c")
    s = lax.axis_index("s")
    _zero_init(zeros_hbm, acc, s)
    plsc.subcore_barrier()
    base_row = c * (EPH // CH) + s * NCH1
    _pipelined_prop(tbl_hbm, acc, srcv_all, dstv_all, rows,
                    sems[:NB], sems[NB:], edges_hbm.at[0], edges_hbm.at[1],
                    base_row, NCH1)
    plsc.subcore_barrier()
    _writeback(acc, out_hbm, c, s)


@functools.partial(
    pl.kernel,
    out_type=jax.ShapeDtypeStruct((NC, NP, 128), F32),
    mesh=_mesh,
    scratch_types=[
        pltpu.VMEM_SHARED((NP, 128), F32),
        pltpu.VMEM((2, CH, 128), F32),
        pltpu.VMEM((2, IB2, CH), jnp.int32),
        pltpu.VMEM((2, IB2, CH), jnp.int32),
        pltpu.SemaphoreType.DMA,
        pltpu.SemaphoreType.DMA,
        pltpu.SemaphoreType.DMA,
        pltpu.SemaphoreType.DMA,
        pltpu.SemaphoreType.DMA,
        pltpu.SemaphoreType.DMA,
    ],
)
def _sc_prop2(edges_hbm, tblA_hbm, tblB_hbm, zeros_hbm, out_hbm,
              acc, rows, srcb, dstb, g0, g1, s0, s1, i0, i1):
    c = lax.axis_index("c")
    s = lax.axis_index("s")
    _zero_init(zeros_hbm, acc, s)
    plsc.subcore_barrier()
    base_row = s * NCH2
    nblk = NCH2 // IB2

    @pl.when(c == 0)
    def _():
        _blocked_prop(tblA_hbm, acc, srcb, dstb, rows, [g0, g1], [s0, s1],
                      [i0, i1], edges_hbm.at[0], edges_hbm.at[1], base_row,
                      nblk, IB2)

    @pl.when(c == 1)
    def _():
        _blocked_prop(tblB_hbm, acc, srcb, dstb, rows, [g0, g1], [s0, s1],
                      [i0, i1], edges_hbm.at[0], edges_hbm.at[1], base_row,
                      nblk, IB2)

    plsc.subcore_barrier()
    _writeback(acc, out_hbm, c, s)


def _tc_hidden_body(p1_ref, xs_ref, dinv_ref, W1_ref, b1_ref, W2_ref,
                    tblA_ref, tblB_ref):
    dinv = dinv_ref[...]
    px = dinv * (p1_ref[0, :N, 0:4] + p1_ref[1, :N, 0:4] + xs_ref[:N, 0:4])
    h1 = jax.nn.relu(
        jnp.dot(px, W1_ref[...], preferred_element_type=F32) + b1_ref[...])
    g = jnp.dot(h1, W2_ref[...], preferred_element_type=F32)
    gs = g * dinv
    pad = ((0, NP - N), (0, 0))
    tblA_ref[...] = jnp.pad(gs[:, :128], pad)
    tblB_ref[...] = jnp.pad(gs[:, 128:], pad)


def _tc_hidden(p1, xs, dinv, W1, b1, W2):
    return pl.pallas_call(
        _tc_hidden_body,
        out_shape=(jax.ShapeDtypeStruct((NP, 128), F32),
                   jax.ShapeDtypeStruct((NP, 128), F32)),
    )(p1, xs, dinv, W1, b1, W2)


def _bn(h, g, b, eps=1e-5):
    m = jnp.mean(h, axis=0)
    v = jnp.var(h, axis=0)
    return (h - m) * lax.rsqrt(v + eps) * g + b


def _elu(h):
    return jnp.where(h > 0, h, jnp.exp(h) - 1.0)


def _tc_head_body(p2_ref, tblA_ref, tblB_ref, dinv_ref, b2_ref, batchT_ref,
                  drug_ref,
                  cW1_ref, cb1_ref, cg1_ref, cbe1_ref, cW2_ref, cb2_ref,
                  dW1_ref, db1_ref, dg1_ref, dbe1_ref, dW2_ref, db2_ref,
                  dg2_ref, dbe2_ref, fW1_ref, fb1_ref, fg1_ref, fbe1_ref,
                  fW2_ref, fb2_ref, fg2_ref, fbe2_ref, fW3_ref, fb3_ref,
                  y_ref):
    dinv = dinv_ref[...]
    agg = jnp.concatenate(
        [p2_ref[0, :N, :] + tblA_ref[:N, :],
         p2_ref[1, :N, :] + tblB_ref[:N, :]], axis=1)
    h2 = jax.nn.relu(dinv * agg + b2_ref[...])

    cols = batchT_ref[...]
    rows = lax.broadcasted_iota(jnp.int32, (B, N), 0)
    maskT = (cols == rows).astype(F32)
    cnt = jnp.sum(maskT, axis=1)
    ssum = jnp.dot(maskT, h2, preferred_element_type=F32)
    pooled = ssum / jnp.maximum(cnt, 1.0)[:, None]

    dh = jax.nn.relu(_bn(
        jnp.dot(drug_ref[...], dW1_ref[...], preferred_element_type=F32)
        + db1_ref[...], dg1_ref[...], dbe1_ref[...]))
    dh = jax.nn.relu(_bn(
        jnp.dot(dh, dW2_ref[...], preferred_element_type=F32)
        + db2_ref[...], dg2_ref[...], dbe2_ref[...]))

    ch = jax.nn.relu(_bn(
        jnp.dot(pooled, cW1_ref[...], preferred_element_type=F32)
        + cb1_ref[...], cg1_ref[...], cbe1_ref[...]))
    ch = jax.nn.relu(
        jnp.dot(ch, cW2_ref[...], preferred_element_type=F32) + cb2_ref[...])

    cat = jnp.concatenate([ch, dh], axis=-1)
    f = _elu(_bn(
        jnp.dot(cat, fW1_ref[...], preferred_element_type=F32)
        + fb1_ref[...], fg1_ref[...], fbe1_ref[...]))
    f = _elu(_bn(
        jnp.dot(f, fW2_ref[...], preferred_element_type=F32)
        + fb2_ref[...], fg2_ref[...], fbe2_ref[...]))
    y_ref[...] = jnp.dot(f, fW3_ref[...], preferred_element_type=F32) \
        + fb3_ref[...]


def _tc_head(p2, tblA, tblB, dinv, b2, batchT, drug, args):
    return pl.pallas_call(
        _tc_head_body,
        out_shape=jax.ShapeDtypeStruct((B, 1), F32),
    )(p2, tblA, tblB, dinv, b2, batchT, drug, *args)


def kernel(x, edge_index, batch, drug, W1, b1, W2, b2, cW1, cb1, cg1, cbe1,
           cW2, cb2, dW1, db1, dg1, dbe1, dW2, db2, dg2, dbe2, fW1, fb1,
           fg1, fbe1, fW2, fb2, fg2, fbe2, fW3, fb3):
    padi = (N + (jnp.arange(EP - E, dtype=jnp.int32) % 128)).astype(jnp.int32)
    padi2 = jnp.broadcast_to(padi, (2, EP - E))
    edges = jnp.concatenate(
        [edge_index.astype(jnp.int32), padi2], axis=1).reshape(2, EP // CH, CH)

    ones16 = jnp.ones((CH, 16), F32)
    zeros16 = jnp.zeros((SL, 16), F32)
    zeros128 = jnp.zeros((SL, 128), F32)

    degp = _sc_deg(edges, ones16, zeros16)
    deg = degp[0, :N] + degp[1, :N] + 1.0
    dinv = lax.rsqrt(deg)[:, None]
    xs = jnp.pad(x * dinv, ((0, NP - N), (0, 12)))
    p1 = _sc_prop1(edges, xs, zeros16)
    tblA, tblB = _tc_hidden(p1, xs, dinv, W1, b1, W2)
    p2 = _sc_prop2(edges, tblA, tblB, zeros128)

    batchT = batch.reshape(1, N).astype(jnp.int32)
    args = (cW1, cb1, cg1, cbe1, cW2, cb2, dW1, db1, dg1, dbe1, dW2, db2,
            dg2, dbe2, fW1, fb1, fg1, fbe1, fW2, fb2, fg2, fbe2, fW3, fb3)
    y = _tc_head(p2, tblA, tblB, dinv, b2, batchT, drug, args)
    return y.reshape(B)


# speedup vs baseline: 1.0669x; 1.0119x over previous
import functools

import jax
import jax.numpy as jnp
from jax import lax
from jax.experimental import pallas as pl
from jax.experimental.pallas import tpu as pltpu
from jax.experimental.pallas import tpu_sc as plsc

N = 10000
E = 320000
B = 128
F32 = jnp.float32

NC, NS = 2, 16
CH = 128
NP = 10240
SL = NP // NS
EP = 327680
EPH = EP // 2

_mesh = plsc.VectorSubcoreMesh(core_axis_name="c", subcore_axis_name="s")
_sc_params = pltpu.CompilerParams(use_tc_tiling_on_sc=False)
_sc_params_nl = pltpu.CompilerParams(use_tc_tiling_on_sc=False,
                                     needs_layout_passes=False)

NB = 4
LAG = 2


def _pipelined_prop(tbl_hbm, acc, srcv_all, dstv_all, rows, gsems, ssems,
                    src_hbm, dst_hbm, base_row, nch):
    pltpu.sync_copy(src_hbm.at[pl.ds(base_row, nch)], srcv_all)
    pltpu.sync_copy(dst_hbm.at[pl.ds(base_row, nch)], dstv_all)

    @pl.loop(0, nch, step=NB)
    def _(i0):
        for b in range(NB):
            i = i0 + b

            @pl.when(i0 > 0)
            def _():
                pltpu.make_async_copy(
                    rows.at[b], acc.at[dstv_all.at[i - NB]], ssems[b]).wait()

            pltpu.async_copy(tbl_hbm.at[srcv_all.at[i]], rows.at[b], gsems[b])

            bs = (b - LAG) % NB
            j = i - LAG

            @pl.when(j >= 0)
            def _():
                pltpu.make_async_copy(
                    tbl_hbm.at[srcv_all.at[j]], rows.at[bs], gsems[bs]).wait()
                pltpu.async_copy(
                    rows.at[bs], acc.at[dstv_all.at[j]], ssems[bs], add=True)

    for k in range(LAG):
        j = nch - LAG + k
        bs = j % NB
        pltpu.make_async_copy(
            tbl_hbm.at[srcv_all.at[j]], rows.at[bs], gsems[bs]).wait()
        pltpu.async_copy(rows.at[bs], acc.at[dstv_all.at[j]], ssems[bs],
                         add=True)
    for b in range(NB):
        pltpu.make_async_copy(
            rows.at[b], acc.at[dstv_all.at[nch - NB + b]], ssems[b]).wait()


IB2 = 16


def _ring_block(tbl_hbm, acc, srcv, dstv, rows, gsems, ssems, ib):
    pltpu.async_copy(tbl_hbm.at[srcv.at[0]], rows.at[0], gsems[0])

    @pl.loop(0, ib, step=2)
    def _(c0):
        for b in range(2):
            c = c0 + b
            nxt = 1 - b
            if b == 0:
                @pl.when((c0 >= 1) & (c0 + 1 < ib))
                def _():
                    pltpu.make_async_copy(
                        rows.at[1], acc.at[dstv.at[c - 1]], ssems[1]).wait()
                    pltpu.async_copy(tbl_hbm.at[srcv.at[c + 1]], rows.at[1],
                                     gsems[1])

                @pl.when(c0 < 1)
                def _():
                    pltpu.async_copy(tbl_hbm.at[srcv.at[c + 1]], rows.at[1],
                                     gsems[1])
            else:
                @pl.when(c0 + 2 < ib)
                def _():
                    pltpu.make_async_copy(
                        rows.at[0], acc.at[dstv.at[c - 1]], ssems[0]).wait()
                    pltpu.async_copy(tbl_hbm.at[srcv.at[c + 1]], rows.at[0],
                                     gsems[0])

            pltpu.make_async_copy(
                tbl_hbm.at[srcv.at[c]], rows.at[b], gsems[b]).wait()
            pltpu.async_copy(rows.at[b], acc.at[dstv.at[c]], ssems[b],
                             add=True)

    pltpu.make_async_copy(rows.at[0], acc.at[dstv.at[ib - 2]],
                          ssems[0]).wait()
    pltpu.make_async_copy(rows.at[1], acc.at[dstv.at[ib - 1]],
                          ssems[1]).wait()


def _blocked_prop(tbl_hbm, acc, srcb, dstb, rows, gsems, ssems, isems,
                  src_hbm, dst_hbm, base_row, nblk, ib):
    def load_blk(k, slot):
        pltpu.async_copy(src_hbm.at[pl.ds(base_row + k * ib, ib)],
                         srcb.at[slot], isems[slot])
        pltpu.async_copy(dst_hbm.at[pl.ds(base_row + k * ib, ib)],
                         dstb.at[slot], isems[slot])

    def wait_blk(k, slot):
        pltpu.make_async_copy(src_hbm.at[pl.ds(base_row + k * ib, ib)],
                              srcb.at[slot], isems[slot]).wait()
        pltpu.make_async_copy(dst_hbm.at[pl.ds(base_row + k * ib, ib)],
                              dstb.at[slot], isems[slot]).wait()

    load_blk(0, 0)
    load_blk(1, 1)

    @pl.loop(0, nblk, step=2)
    def _(k0):
        for kb in range(2):
            k = k0 + kb
            wait_blk(k, kb)
            _ring_block(tbl_hbm, acc, srcb.at[kb], dstb.at[kb], rows,
                        gsems, ssems, ib)

            @pl.when(k + 2 < nblk)
            def _():
                load_blk(k + 2, kb)


def _zero_init(zeros_hbm, acc, s):
    pltpu.sync_copy(zeros_hbm, acc.at[pl.ds(s * SL, SL)])


def _writeback(acc, out_hbm, c, s):
    pltpu.sync_copy(acc.at[pl.ds(s * SL, SL)], out_hbm.at[c, pl.ds(s * SL, SL)])


NCH1 = EPH // NS // CH
NCH2 = EP // NS // CH
_SEMS4 = [pltpu.SemaphoreType.DMA] * NB


@functools.partial(
    pl.kernel,
    out_type=jax.ShapeDtypeStruct((NC, NP), F32),
    mesh=_mesh,
    scratch_types=[
        pltpu.VMEM_SHARED((NP, 16), F32),
        pltpu.VMEM((CH, 16), F32),
        pltpu.VMEM((NCH1, CH), jnp.int32),
        pltpu.VMEM((SL, 16), F32),
        pltpu.VMEM((SL,), F32),
    ] + _SEMS4,
    compiler_params=_sc_params_nl,
)
def _sc_deg(edges_hbm, ones_hbm, zeros_hbm, out_hbm, acc, ones_v, dstv_all,
            degbuf, packed, *ssems):
    c = lax.axis_index("c")
    s = lax.axis_index("s")
    pltpu.sync_copy(ones_hbm, ones_v)
    _zero_init(zeros_hbm, acc, s)
    plsc.subcore_barrier()
    base_row = c * (EPH // CH) + s * NCH1
    pltpu.sync_copy(edges_hbm.at[1, pl.ds(base_row, NCH1)], dstv_all)

    @pl.loop(0, NCH1, step=NB)
    def _(i0):
        for b in range(NB):
            i = i0 + b

            @pl.when(i0 > 0)
            def _():
                pltpu.make_async_copy(
                    ones_v, acc.at[dstv_all.at[i - NB]], ssems[b]).wait()

            pltpu.async_copy(ones_v, acc.at[dstv_all.at[i]], ssems[b],
                             add=True)

    for b in range(NB):
        pltpu.make_async_copy(
            ones_v, acc.at[dstv_all.at[NCH1 - NB + b]], ssems[b]).wait()

    plsc.subcore_barrier()
    pltpu.sync_copy(acc.at[pl.ds(s * SL, SL)], degbuf)
    lane = lax.iota(jnp.int32, 16)
    zero16 = lane * 0

    @pl.loop(0, SL // 16)
    def _(r0):
        vals = plsc.load_gather(degbuf, [r0 * 16 + lane, zero16])
        packed[pl.ds(r0 * 16, 16)] = vals

    pltpu.sync_copy(packed, out_hbm.at[c, pl.ds(s * SL, SL)])


@functools.partial(
    pl.kernel,
    out_type=jax.ShapeDtypeStruct((NC, 4, NP), F32),
    mesh=_mesh,
    scratch_types=[
        pltpu.VMEM_SHARED((NP, 16), F32),
        pltpu.VMEM((NB, CH, 16), F32),
        pltpu.VMEM((NCH1, CH), jnp.int32),
        pltpu.VMEM((NCH1, CH), jnp.int32),
        pltpu.VMEM((SL, 16), F32),
        pltpu.VMEM((4 * SL,), F32),
    ] + _SEMS4 + _SEMS4,
    compiler_params=_sc_params_nl,
)
def _sc_prop1(edges_hbm, tbl_hbm, zeros_hbm, out_hbm, acc,
              rows, srcv_all, dstv_all, accbuf, packed, *sems):
    c = lax.axis_index("c")
    s = lax.axis_index("s")
    _zero_init(zeros_hbm, acc, s)
    plsc.subcore_barrier()
    base_row = c * (EPH // CH) + s * NCH1
    _pipelined_prop(tbl_hbm, acc, srcv_all, dstv_all, rows,
                    sems[:NB], sems[NB:], edges_hbm.at[0], edges_hbm.at[1],
                    base_row, NCH1)
    plsc.subcore_barrier()
    pltpu.sync_copy(acc.at[pl.ds(s * SL, SL)], accbuf)
    lane = lax.iota(jnp.int32, 16)

    @pl.loop(0, SL // 16)
    def _(r0):
        for j in range(4):
            vals = plsc.load_gather(accbuf, [r0 * 16 + lane, lane * 0 + j])
            packed[pl.ds(j * SL + r0 * 16, 16)] = vals

    for j in range(4):
        pltpu.sync_copy(packed.at[pl.ds(j * SL, SL)],
                        out_hbm.at[c, j, pl.ds(s * SL, SL)])


@functools.partial(
    pl.kernel,
    out_type=jax.ShapeDtypeStruct((NC, NP, 128), F32),
    mesh=_mesh,
    scratch_types=[
        pltpu.VMEM_SHARED((NP, 128), F32),
        pltpu.VMEM((2, CH, 128), F32),
        pltpu.VMEM((2, IB2, CH), jnp.int32),
        pltpu.VMEM((2, IB2, CH), jnp.int32),
        pltpu.SemaphoreType.DMA,
        pltpu.SemaphoreType.DMA,
        pltpu.SemaphoreType.DMA,
        pltpu.SemaphoreType.DMA,
        pltpu.SemaphoreType.DMA,
        pltpu.SemaphoreType.DMA,
    ],
)
def _sc_prop2(edges_hbm, tblA_hbm, tblB_hbm, zeros_hbm, out_hbm,
              acc, rows, srcb, dstb, g0, g1, s0, s1, i0, i1):
    c = lax.axis_index("c")
    s = lax.axis_index("s")
    _zero_init(zeros_hbm, acc, s)
    plsc.subcore_barrier()
    base_row = s * NCH2
    nblk = NCH2 // IB2

    @pl.when(c == 0)
    def _():
        _blocked_prop(tblA_hbm, acc, srcb, dstb, rows, [g0, g1], [s0, s1],
                      [i0, i1], edges_hbm.at[0], edges_hbm.at[1], base_row,
                      nblk, IB2)

    @pl.when(c == 1)
    def _():
        _blocked_prop(tblB_hbm, acc, srcb, dstb, rows, [g0, g1], [s0, s1],
                      [i0, i1], edges_hbm.at[0], edges_hbm.at[1], base_row,
                      nblk, IB2)

    plsc.subcore_barrier()
    _writeback(acc, out_hbm, c, s)


def _tc_hidden_body(p1_ref, xs_ref, dinv_ref, W1_ref, b1_ref, W2_ref,
                    tblA_ref, tblB_ref):
    dinv = dinv_ref[...]
    px = dinv * (p1_ref[...] + xs_ref[:N, 0:4])
    h1 = jax.nn.relu(
        jnp.dot(px, W1_ref[...], preferred_element_type=F32) + b1_ref[...])
    g = jnp.dot(h1, W2_ref[...], preferred_element_type=F32)
    gs = g * dinv
    pad = ((0, NP - N), (0, 0))
    tblA_ref[...] = jnp.pad(gs[:, :128], pad)
    tblB_ref[...] = jnp.pad(gs[:, 128:], pad)


def _tc_hidden(p1, xs, dinv, W1, b1, W2):
    return pl.pallas_call(
        _tc_hidden_body,
        out_shape=(jax.ShapeDtypeStruct((NP, 128), F32),
                   jax.ShapeDtypeStruct((NP, 128), F32)),
    )(p1, xs, dinv, W1, b1, W2)


def _bn(h, g, b, eps=1e-5):
    m = jnp.mean(h, axis=0)
    v = jnp.var(h, axis=0)
    return (h - m) * lax.rsqrt(v + eps) * g + b


def _elu(h):
    return jnp.where(h > 0, h, jnp.exp(h) - 1.0)


def _tc_head_body(p2_ref, tblA_ref, tblB_ref, dinv_ref, b2_ref, batchT_ref,
                  drug_ref,
                  cW1_ref, cb1_ref, cg1_ref, cbe1_ref, cW2_ref, cb2_ref,
                  dW1_ref, db1_ref, dg1_ref, dbe1_ref, dW2_ref, db2_ref,
                  dg2_ref, dbe2_ref, fW1_ref, fb1_ref, fg1_ref, fbe1_ref,
                  fW2_ref, fb2_ref, fg2_ref, fbe2_ref, fW3_ref, fb3_ref,
                  y_ref):
    dinv = dinv_ref[...]
    agg = jnp.concatenate(
        [p2_ref[0, :N, :] + tblA_ref[:N, :],
         p2_ref[1, :N, :] + tblB_ref[:N, :]], axis=1)
    h2 = jax.nn.relu(dinv * agg + b2_ref[...])

    cols = batchT_ref[...]
    rows = lax.broadcasted_iota(jnp.int32, (B, N), 0)
    maskT = (cols == rows).astype(F32)
    cnt = jnp.sum(maskT, axis=1)
    ssum = jnp.dot(maskT, h2, preferred_element_type=F32)
    pooled = ssum / jnp.maximum(cnt, 1.0)[:, None]

    dh = jax.nn.relu(_bn(
        jnp.dot(drug_ref[...], dW1_ref[...], preferred_element_type=F32)
        + db1_ref[...], dg1_ref[...], dbe1_ref[...]))
    dh = jax.nn.relu(_bn(
        jnp.dot(dh, dW2_ref[...], preferred_element_type=F32)
        + db2_ref[...], dg2_ref[...], dbe2_ref[...]))

    ch = jax.nn.relu(_bn(
        jnp.dot(pooled, cW1_ref[...], preferred_element_type=F32)
        + cb1_ref[...], cg1_ref[...], cbe1_ref[...]))
    ch = jax.nn.relu(
        jnp.dot(ch, cW2_ref[...], preferred_element_type=F32) + cb2_ref[...])

    cat = jnp.concatenate([ch, dh], axis=-1)
    f = _elu(_bn(
        jnp.dot(cat, fW1_ref[...], preferred_element_type=F32)
        + fb1_ref[...], fg1_ref[...], fbe1_ref[...]))
    f = _elu(_bn(
        jnp.dot(f, fW2_ref[...], preferred_element_type=F32)
        + fb2_ref[...], fg2_ref[...], fbe2_ref[...]))
    y_ref[...] = jnp.dot(f, fW3_ref[...], preferred_element_type=F32) \
        + fb3_ref[...]


def _tc_head(p2, tblA, tblB, dinv, b2, batchT, drug, args):
    return pl.pallas_call(
        _tc_head_body,
        out_shape=jax.ShapeDtypeStruct((B, 1), F32),
    )(p2, tblA, tblB, dinv, b2, batchT, drug, *args)


def kernel(x, edge_index, batch, drug, W1, b1, W2, b2, cW1, cb1, cg1, cbe1,
           cW2, cb2, dW1, db1, dg1, dbe1, dW2, db2, dg2, dbe2, fW1, fb1,
           fg1, fbe1, fW2, fb2, fg2, fbe2, fW3, fb3):
    padi = (N + (jnp.arange(EP - E, dtype=jnp.int32) % 128)).astype(jnp.int32)
    padi2 = jnp.broadcast_to(padi, (2, EP - E))
    edges = jnp.concatenate(
        [edge_index.astype(jnp.int32), padi2], axis=1).reshape(2, EP // CH, CH)

    ones16 = jnp.ones((CH, 16), F32)
    zeros16 = jnp.zeros((SL, 16), F32)
    zeros128 = jnp.zeros((SL, 128), F32)

    degp = _sc_deg(edges, ones16, zeros16)
    deg = degp[0, :N] + degp[1, :N] + 1.0
    dinv = lax.rsqrt(deg)[:, None]
    xs = jnp.pad(x * dinv, ((0, NP - N), (0, 12)))
    p1t = _sc_prop1(edges, xs, zeros16)
    p1sum = jnp.transpose(p1t[0, :, :N] + p1t[1, :, :N])
    tblA, tblB = _tc_hidden(p1sum, xs, dinv, W1, b1, W2)
    p2 = _sc_prop2(edges, tblA, tblB, zeros128)

    batchT = batch.reshape(1, N).astype(jnp.int32)
    args = (cW1, cb1, cg1, cbe1, cW2, cb2, dW1, db1, dg1, dbe1, dW2, db2,
            dg2, dbe2, fW1, fb1, fg1, fbe1, fW2, fb2, fg2, fbe2, fW3, fb3)
    y = _tc_head(p2, tblA, tblB, dinv, b2, batchT, drug, args)
    return y.reshape(B)
